# baseline (speedup 1.0000x reference)
import jax
import jax.numpy as jnp
from jax.experimental import pallas as pl
from jax.experimental.pallas import tpu as pltpu

USER_N = 6000
ITEM_N = 4000
NTOT = USER_N + ITEM_N
LAT = 32
HYP = 128
BM = 200
RBLKS = NTOT // BM
UBLKS = USER_N // BM

_F32 = jnp.float32


def _dotT(a, b):
    return jax.lax.dot_general(a, b, (((0,), (0,)), ((), ())),
                               preferred_element_type=_F32)


def _body(adj_ref, emb_ref, embblk_ref, uH_ref, iH_ref,
          tem_ref, h_ref, eo_ref,
          e1_s, Ku_s, Ki_s, Pu_s, Pi_s, Lu_s, Li_s):
    l = pl.program_id(0)
    r = pl.program_id(1)

    @pl.when((l == 0) & (r == 0))
    def _prep():
        uE = emb_ref[:USER_N, :]
        iE = emb_ref[USER_N:, :]
        Gu = _dotT(uE, uE)
        Gi = _dotT(iE, iE)
        Ku_s[...] = jnp.dot(uH_ref[...], _dotT(uH_ref[...], Gu),
                            preferred_element_type=_F32)
        Ki_s[...] = jnp.dot(iH_ref[...], _dotT(iH_ref[...], Gi),
                            preferred_element_type=_F32)
        Pu_s[...] = jnp.zeros_like(Pu_s)
        Pi_s[...] = jnp.zeros_like(Pi_s)

    a = adj_ref[...]
    tem = jax.lax.cond(
        l == 0,
        lambda: jnp.dot(a, emb_ref[...], preferred_element_type=_F32),
        lambda: jnp.dot(a, e1_s[...], preferred_element_type=_F32))
    K = jnp.where(l == 0,
                  jnp.where(r < UBLKS, Ku_s[...], Ki_s[...]),
                  jnp.where(r < UBLKS, Lu_s[...], Li_s[...]))
    eblk = embblk_ref[...]
    h = jnp.dot(eblk, K, preferred_element_type=_F32)
    lat = tem + h
    tem_ref[...] = tem.reshape(1, BM, LAT)
    h_ref[...] = h.reshape(1, BM, LAT)

    @pl.when(l == 0)
    def _layer1_epi():
        e1_s[pl.ds(r * BM, BM), :] = lat
        eo_ref[...] = lat.reshape(1, BM, LAT)
        contrib = _dotT(eblk, lat)

        @pl.when(r < UBLKS)
        def _():
            Pu_s[...] += contrib

        @pl.when(r >= UBLKS)
        def _():
            Pi_s[...] += contrib

    @pl.when((l == 0) & (r == RBLKS - 1))
    def _mid():
        Lu_s[...] = jnp.dot(uH_ref[...], _dotT(uH_ref[...], Pu_s[...]),
                            preferred_element_type=_F32)
        Li_s[...] = jnp.dot(iH_ref[...], _dotT(iH_ref[...], Pi_s[...]),
                            preferred_element_type=_F32)

    @pl.when(l == 1)
    def _layer2_epi():
        out = eblk + e1_s[pl.ds(r * BM, BM), :] + lat
        eo_ref[...] = out.reshape(1, BM, LAT)


_call = pl.pallas_call(
    _body,
    grid=(2, RBLKS),
    in_specs=[
        pl.BlockSpec((BM, NTOT), lambda l, r: (r, 0)),
        pl.BlockSpec((NTOT, LAT), lambda l, r: (0, 0)),
        pl.BlockSpec((BM, LAT), lambda l, r: (r, 0)),
        pl.BlockSpec((LAT, HYP), lambda l, r: (0, 0)),
        pl.BlockSpec((LAT, HYP), lambda l, r: (0, 0)),
    ],
    out_specs=[
        pl.BlockSpec((1, BM, LAT), lambda l, r: (l, r, 0)),
        pl.BlockSpec((1, BM, LAT), lambda l, r: (l, r, 0)),
        pl.BlockSpec((1, BM, LAT), lambda l, r: (l, r, 0)),
    ],
    out_shape=[
        jax.ShapeDtypeStruct((2, NTOT, LAT), _F32),
        jax.ShapeDtypeStruct((2, NTOT, LAT), _F32),
        jax.ShapeDtypeStruct((2, NTOT, LAT), _F32),
    ],
    scratch_shapes=[
        pltpu.VMEM((NTOT, LAT), _F32),
        pltpu.VMEM((LAT, LAT), _F32),
        pltpu.VMEM((LAT, LAT), _F32),
        pltpu.VMEM((LAT, LAT), _F32),
        pltpu.VMEM((LAT, LAT), _F32),
        pltpu.VMEM((LAT, LAT), _F32),
        pltpu.VMEM((LAT, LAT), _F32),
    ],
    compiler_params=pltpu.CompilerParams(
        dimension_semantics=("arbitrary", "arbitrary"),
        vmem_limit_bytes=40 * 1024 * 1024),
)


def kernel(adj, keepRate, uEmbeds, iEmbeds, uHyper, iHyper):
    del keepRate
    emb = jnp.concatenate([uEmbeds, iEmbeds], axis=0)
    tem, h, eo = _call(adj, emb, emb, uHyper, iHyper)
    return (eo[1], tem[0], tem[1], h[0], h[1])


# speedup vs baseline: 1.0927x; 1.0927x over previous
import jax
import jax.numpy as jnp
from jax.experimental import pallas as pl
from jax.experimental.pallas import tpu as pltpu

USER_N = 6000
ITEM_N = 4000
NTOT = USER_N + ITEM_N
LAT = 32
HYP = 128
BM = 200
RBLKS = NTOT // BM
UBLKS = USER_N // BM

_F32 = jnp.float32


def _dotT(a, b):
    return jax.lax.dot_general(a, b, (((0,), (0,)), ((), ())),
                               preferred_element_type=_F32)


def _prep_body(emb_ref, uH_ref, iH_ref, Ku_ref, Ki_ref):
    uE = emb_ref[:USER_N, :]
    iE = emb_ref[USER_N:, :]
    Gu = _dotT(uE, uE)
    Gi = _dotT(iE, iE)
    Ku_ref[...] = jnp.dot(uH_ref[...], _dotT(uH_ref[...], Gu),
                          preferred_element_type=_F32)
    Ki_ref[...] = jnp.dot(iH_ref[...], _dotT(iH_ref[...], Gi),
                          preferred_element_type=_F32)


_prep = pl.pallas_call(
    _prep_body,
    out_shape=[jax.ShapeDtypeStruct((LAT, LAT), _F32),
               jax.ShapeDtypeStruct((LAT, LAT), _F32)],
)


def _big1_body(adj_ref, emb_ref, embblk_ref, Ku_ref, Ki_ref,
               tem_ref, h_ref, e1_ref, c_ref):
    r = pl.program_id(0)
    tem = jnp.dot(adj_ref[...], emb_ref[...], preferred_element_type=_F32)
    K = jnp.where(r < UBLKS, Ku_ref[...], Ki_ref[...])
    eblk = embblk_ref[...]
    h = jnp.dot(eblk, K, preferred_element_type=_F32)
    e1 = tem + h
    tem_ref[...] = tem
    h_ref[...] = h
    e1_ref[...] = e1
    c_ref[...] = _dotT(eblk, e1).reshape(1, LAT, LAT)


_big1 = pl.pallas_call(
    _big1_body,
    grid=(RBLKS,),
    in_specs=[
        pl.BlockSpec((BM, NTOT), lambda r: (r, 0)),
        pl.BlockSpec((NTOT, LAT), lambda r: (0, 0)),
        pl.BlockSpec((BM, LAT), lambda r: (r, 0)),
        pl.BlockSpec((LAT, LAT), lambda r: (0, 0)),
        pl.BlockSpec((LAT, LAT), lambda r: (0, 0)),
    ],
    out_specs=[
        pl.BlockSpec((BM, LAT), lambda r: (r, 0)),
        pl.BlockSpec((BM, LAT), lambda r: (r, 0)),
        pl.BlockSpec((BM, LAT), lambda r: (r, 0)),
        pl.BlockSpec((1, LAT, LAT), lambda r: (r, 0, 0)),
    ],
    out_shape=[
        jax.ShapeDtypeStruct((NTOT, LAT), _F32),
        jax.ShapeDtypeStruct((NTOT, LAT), _F32),
        jax.ShapeDtypeStruct((NTOT, LAT), _F32),
        jax.ShapeDtypeStruct((RBLKS, LAT, LAT), _F32),
    ],
    compiler_params=pltpu.CompilerParams(
        dimension_semantics=("parallel",)),
)


def _mid_body(c_ref, uH_ref, iH_ref, Lu_ref, Li_ref):
    Pu = jnp.sum(c_ref[:UBLKS], axis=0)
    Pi = jnp.sum(c_ref[UBLKS:], axis=0)
    Lu_ref[...] = jnp.dot(uH_ref[...], _dotT(uH_ref[...], Pu),
                          preferred_element_type=_F32)
    Li_ref[...] = jnp.dot(iH_ref[...], _dotT(iH_ref[...], Pi),
                          preferred_element_type=_F32)


_mid = pl.pallas_call(
    _mid_body,
    out_shape=[jax.ShapeDtypeStruct((LAT, LAT), _F32),
               jax.ShapeDtypeStruct((LAT, LAT), _F32)],
)


def _big2_body(adj_ref, e1_ref, embblk_ref, e1blk_ref, Lu_ref, Li_ref,
               tem2_ref, h2_ref, out_ref):
    r = pl.program_id(0)
    tem2 = jnp.dot(adj_ref[...], e1_ref[...], preferred_element_type=_F32)
    L = jnp.where(r < UBLKS, Lu_ref[...], Li_ref[...])
    eblk = embblk_ref[...]
    h2 = jnp.dot(eblk, L, preferred_element_type=_F32)
    tem2_ref[...] = tem2
    h2_ref[...] = h2
    out_ref[...] = eblk + e1blk_ref[...] + tem2 + h2


_big2 = pl.pallas_call(
    _big2_body,
    grid=(RBLKS,),
    in_specs=[
        pl.BlockSpec((BM, NTOT), lambda r: (r, 0)),
        pl.BlockSpec((NTOT, LAT), lambda r: (0, 0)),
        pl.BlockSpec((BM, LAT), lambda r: (r, 0)),
        pl.BlockSpec((BM, LAT), lambda r: (r, 0)),
        pl.BlockSpec((LAT, LAT), lambda r: (0, 0)),
        pl.BlockSpec((LAT, LAT), lambda r: (0, 0)),
    ],
    out_specs=[
        pl.BlockSpec((BM, LAT), lambda r: (r, 0)),
        pl.BlockSpec((BM, LAT), lambda r: (r, 0)),
        pl.BlockSpec((BM, LAT), lambda r: (r, 0)),
    ],
    out_shape=[
        jax.ShapeDtypeStruct((NTOT, LAT), _F32),
        jax.ShapeDtypeStruct((NTOT, LAT), _F32),
        jax.ShapeDtypeStruct((NTOT, LAT), _F32),
    ],
    compiler_params=pltpu.CompilerParams(
        dimension_semantics=("parallel",)),
)


def kernel(adj, keepRate, uEmbeds, iEmbeds, uHyper, iHyper):
    del keepRate
    emb = jnp.concatenate([uEmbeds, iEmbeds], axis=0)
    Ku, Ki = _prep(emb, uHyper, iHyper)
    tem1, h1, e1, c = _big1(adj, emb, emb, Ku, Ki)
    Lu, Li = _mid(c, uHyper, iHyper)
    tem2, h2, out = _big2(adj, e1, emb, e1, Lu, Li)
    return (out, tem1, tem2, h1, h2)


# speedup vs baseline: 1.1040x; 1.0103x over previous
import jax
import jax.numpy as jnp
from jax.experimental import pallas as pl
from jax.experimental.pallas import tpu as pltpu

USER_N = 6000
ITEM_N = 4000
NTOT = USER_N + ITEM_N
LAT = 32
HYP = 128
BM = 400
RBLKS = NTOT // BM
UBLKS = USER_N // BM
VLIM = 100 * 1024 * 1024

_F32 = jnp.float32


def _dotT(a, b):
    return jax.lax.dot_general(a, b, (((0,), (0,)), ((), ())),
                               preferred_element_type=_F32)


def _layer1_body(adj_ref, emb_ref, embblk_ref, uH_ref, iH_ref,
                 tem_ref, h_ref, e1_ref, Lu_ref, Li_ref,
                 Ku_s, Ki_s, Pu_s, Pi_s):
    r = pl.program_id(0)

    @pl.when(r == 0)
    def _prep():
        uE = emb_ref[:USER_N, :]
        iE = emb_ref[USER_N:, :]
        Gu = _dotT(uE, uE)
        Gi = _dotT(iE, iE)
        Ku_s[...] = jnp.dot(uH_ref[...], _dotT(uH_ref[...], Gu),
                            preferred_element_type=_F32)
        Ki_s[...] = jnp.dot(iH_ref[...], _dotT(iH_ref[...], Gi),
                            preferred_element_type=_F32)
        Pu_s[...] = jnp.zeros_like(Pu_s)
        Pi_s[...] = jnp.zeros_like(Pi_s)

    tem = jnp.dot(adj_ref[...], emb_ref[...], preferred_element_type=_F32)
    eblk = embblk_ref[...]
    K = jnp.where(r < UBLKS, Ku_s[...], Ki_s[...])
    h = jnp.dot(eblk, K, preferred_element_type=_F32)
    e1 = tem + h
    tem_ref[...] = tem
    h_ref[...] = h
    e1_ref[...] = e1
    contrib = _dotT(eblk, e1)

    @pl.when(r < UBLKS)
    def _accu():
        Pu_s[...] += contrib

    @pl.when(r >= UBLKS)
    def _acci():
        Pi_s[...] += contrib

    @pl.when(r == RBLKS - 1)
    def _fin():
        Lu_ref[...] = jnp.dot(uH_ref[...], _dotT(uH_ref[...], Pu_s[...]),
                              preferred_element_type=_F32)
        Li_ref[...] = jnp.dot(iH_ref[...], _dotT(iH_ref[...], Pi_s[...]),
                              preferred_element_type=_F32)


def _layer2_body(adj_ref, e1_ref, embblk_ref, e1blk_ref, Lu_ref, Li_ref,
                 tem2_ref, h2_ref, out_ref):
    r = pl.program_id(0)
    tem2 = jnp.dot(adj_ref[...], e1_ref[...], preferred_element_type=_F32)
    L = jnp.where(r < UBLKS, Lu_ref[...], Li_ref[...])
    eblk = embblk_ref[...]
    h2 = jnp.dot(eblk, L, preferred_element_type=_F32)
    tem2_ref[...] = tem2
    h2_ref[...] = h2
    out_ref[...] = eblk + e1blk_ref[...] + tem2 + h2


def _row_spec():
    return pl.BlockSpec((BM, NTOT), lambda r: (r, 0))


def _full_spec(shape):
    return pl.BlockSpec(shape, lambda r: (0, 0))


def _blk_spec():
    return pl.BlockSpec((BM, LAT), lambda r: (r, 0))


_layer1 = pl.pallas_call(
    _layer1_body,
    grid=(RBLKS,),
    in_specs=[
        _row_spec(),
        _full_spec((NTOT, LAT)),
        _blk_spec(),
        _full_spec((LAT, HYP)),
        _full_spec((LAT, HYP)),
    ],
    out_specs=[
        _blk_spec(),
        _blk_spec(),
        _blk_spec(),
        _full_spec((LAT, LAT)),
        _full_spec((LAT, LAT)),
    ],
    out_shape=[
        jax.ShapeDtypeStruct((NTOT, LAT), _F32),
        jax.ShapeDtypeStruct((NTOT, LAT), _F32),
        jax.ShapeDtypeStruct((NTOT, LAT), _F32),
        jax.ShapeDtypeStruct((LAT, LAT), _F32),
        jax.ShapeDtypeStruct((LAT, LAT), _F32),
    ],
    scratch_shapes=[pltpu.VMEM((LAT, LAT), _F32) for _ in range(4)],
    compiler_params=pltpu.CompilerParams(
        dimension_semantics=("arbitrary",),
        vmem_limit_bytes=VLIM),
)

_layer2 = pl.pallas_call(
    _layer2_body,
    grid=(RBLKS,),
    in_specs=[
        _row_spec(),
        _full_spec((NTOT, LAT)),
        _blk_spec(),
        _blk_spec(),
        _full_spec((LAT, LAT)),
        _full_spec((LAT, LAT)),
    ],
    out_specs=[_blk_spec(), _blk_spec(), _blk_spec()],
    out_shape=[
        jax.ShapeDtypeStruct((NTOT, LAT), _F32),
        jax.ShapeDtypeStruct((NTOT, LAT), _F32),
        jax.ShapeDtypeStruct((NTOT, LAT), _F32),
    ],
    compiler_params=pltpu.CompilerParams(
        dimension_semantics=("arbitrary",),
        vmem_limit_bytes=VLIM),
)


def kernel(adj, keepRate, uEmbeds, iEmbeds, uHyper, iHyper):
    del keepRate
    emb = jnp.concatenate([uEmbeds, iEmbeds], axis=0)
    tem1, h1, e1, Lu, Li = _layer1(adj, emb, emb, uHyper, iHyper)
    tem2, h2, out = _layer2(adj, e1, emb, e1, Lu, Li)
    return (out, tem1, tem2, h1, h2)


# speedup vs baseline: 1.1290x; 1.0226x over previous
import jax
import jax.numpy as jnp
from jax.experimental import pallas as pl
from jax.experimental.pallas import tpu as pltpu

USER_N = 6000
ITEM_N = 4000
NTOT = USER_N + ITEM_N
LAT = 32
HYP = 128
BM = 200
HBLKS = NTOT // (2 * BM)
HALF = NTOT // 2
UB_B = (USER_N - HALF) // BM
VLIM = 100 * 1024 * 1024

_F32 = jnp.float32


def _dotT(a, b):
    return jax.lax.dot_general(a, b, (((0,), (0,)), ((), ())),
                               preferred_element_type=_F32)


def _layer1_body(adjA_ref, adjB_ref, emb_ref, embA_ref, embB_ref,
                 uH_ref, iH_ref,
                 temA_ref, temB_ref, hA_ref, hB_ref, e1A_ref, e1B_ref,
                 Lu_ref, Li_ref,
                 Ku_s, Ki_s, Pu_s, Pi_s):
    r = pl.program_id(0)

    @pl.when(r == 0)
    def _prep():
        uE = emb_ref[:USER_N, :]
        iE = emb_ref[USER_N:, :]
        Gu = _dotT(uE, uE)
        Gi = _dotT(iE, iE)
        Ku_s[...] = jnp.dot(uH_ref[...], _dotT(uH_ref[...], Gu),
                            preferred_element_type=_F32)
        Ki_s[...] = jnp.dot(iH_ref[...], _dotT(iH_ref[...], Gi),
                            preferred_element_type=_F32)
        Pu_s[...] = jnp.zeros_like(Pu_s)
        Pi_s[...] = jnp.zeros_like(Pi_s)

    emb = emb_ref[...]
    temA = jnp.dot(adjA_ref[...], emb, preferred_element_type=_F32)
    temB = jnp.dot(adjB_ref[...], emb, preferred_element_type=_F32)
    eA = embA_ref[...]
    eB = embB_ref[...]
    hA = jnp.dot(eA, Ku_s[...], preferred_element_type=_F32)
    KB = jnp.where(r < UB_B, Ku_s[...], Ki_s[...])
    hB = jnp.dot(eB, KB, preferred_element_type=_F32)
    e1A = temA + hA
    e1B = temB + hB
    temA_ref[...] = temA
    temB_ref[...] = temB
    hA_ref[...] = hA
    hB_ref[...] = hB
    e1A_ref[...] = e1A
    e1B_ref[...] = e1B
    cA = _dotT(eA, e1A)
    cB = _dotT(eB, e1B)
    Pu_s[...] += cA

    @pl.when(r < UB_B)
    def _bu():
        Pu_s[...] += cB

    @pl.when(r >= UB_B)
    def _bi():
        Pi_s[...] += cB

    @pl.when(r == HBLKS - 1)
    def _fin():
        Lu_ref[...] = jnp.dot(uH_ref[...], _dotT(uH_ref[...], Pu_s[...]),
                              preferred_element_type=_F32)
        Li_ref[...] = jnp.dot(iH_ref[...], _dotT(iH_ref[...], Pi_s[...]),
                              preferred_element_type=_F32)


def _layer2_body(adjA_ref, adjB_ref, e1T_ref, e1B_ref,
                 embA_ref, embB_ref, e1blkA_ref, e1blkB_ref,
                 Lu_ref, Li_ref,
                 tem2A_ref, tem2B_ref, h2A_ref, h2B_ref,
                 outA_ref, outB_ref):
    r = pl.program_id(0)
    e1T = e1T_ref[...]
    e1Bot = e1B_ref[...]

    def big(a_ref):
        a = a_ref[...]
        return (jnp.dot(a[:, :HALF], e1T, preferred_element_type=_F32) +
                jnp.dot(a[:, HALF:], e1Bot, preferred_element_type=_F32))

    tem2A = big(adjA_ref)
    tem2B = big(adjB_ref)
    eA = embA_ref[...]
    eB = embB_ref[...]
    h2A = jnp.dot(eA, Lu_ref[...], preferred_element_type=_F32)
    LB = jnp.where(r < UB_B, Lu_ref[...], Li_ref[...])
    h2B = jnp.dot(eB, LB, preferred_element_type=_F32)
    tem2A_ref[...] = tem2A
    tem2B_ref[...] = tem2B
    h2A_ref[...] = h2A
    h2B_ref[...] = h2B
    outA_ref[...] = eA + e1blkA_ref[...] + tem2A + h2A
    outB_ref[...] = eB + e1blkB_ref[...] + tem2B + h2B


def _rowA():
    return pl.BlockSpec((BM, NTOT), lambda r: (r, 0))


def _rowB():
    return pl.BlockSpec((BM, NTOT), lambda r: (r + HBLKS, 0))


def _fullc(shape):
    return pl.BlockSpec(shape, lambda r: tuple(0 for _ in shape))


def _blkA():
    return pl.BlockSpec((BM, LAT), lambda r: (r, 0))


def _blkB():
    return pl.BlockSpec((BM, LAT), lambda r: (r + HBLKS, 0))


def _blkH():
    return pl.BlockSpec((BM, LAT), lambda r: (r, 0))


_half_out = jax.ShapeDtypeStruct((HALF, LAT), _F32)
_ll_out = jax.ShapeDtypeStruct((LAT, LAT), _F32)

_layer1 = pl.pallas_call(
    _layer1_body,
    grid=(HBLKS,),
    in_specs=[
        _rowA(),
        _rowB(),
        _fullc((NTOT, LAT)),
        _blkA(),
        _blkB(),
        _fullc((LAT, HYP)),
        _fullc((LAT, HYP)),
    ],
    out_specs=[
        _blkH(), _blkH(),
        _blkH(), _blkH(),
        _blkH(), _blkH(),
        _fullc((LAT, LAT)),
        _fullc((LAT, LAT)),
    ],
    out_shape=[_half_out, _half_out, _half_out, _half_out,
               _half_out, _half_out, _ll_out, _ll_out],
    scratch_shapes=[pltpu.VMEM((LAT, LAT), _F32) for _ in range(4)],
    compiler_params=pltpu.CompilerParams(
        dimension_semantics=("arbitrary",),
        vmem_limit_bytes=VLIM),
)

_layer2 = pl.pallas_call(
    _layer2_body,
    grid=(HBLKS,),
    in_specs=[
        _rowA(),
        _rowB(),
        _fullc((HALF, LAT)),
        _fullc((HALF, LAT)),
        _blkA(),
        _blkB(),
        _blkH(),
        _blkH(),
        _fullc((LAT, LAT)),
        _fullc((LAT, LAT)),
    ],
    out_specs=[_blkH(), _blkH(), _blkH(), _blkH(), _blkH(), _blkH()],
    out_shape=[_half_out, _half_out, _half_out, _half_out,
               _half_out, _half_out],
    compiler_params=pltpu.CompilerParams(
        dimension_semantics=("arbitrary",),
        vmem_limit_bytes=VLIM),
)


def kernel(adj, keepRate, uEmbeds, iEmbeds, uHyper, iHyper):
    del keepRate
    emb = jnp.concatenate([uEmbeds, iEmbeds], axis=0)
    (temT, temB, hT, hB, e1T, e1B, Lu, Li) = _layer1(
        adj, adj, emb, emb, emb, uHyper, iHyper)
    (tem2T, tem2B, h2T, h2B, outT, outB) = _layer2(
        adj, adj, e1T, e1B, emb, emb, e1T, e1B, Lu, Li)
    cat = lambda t, b: jnp.concatenate([t, b], axis=0)
    return (cat(outT, outB), cat(temT, temB), cat(tem2T, tem2B),
            cat(hT, hB), cat(h2T, h2B))
